# baseline (speedup 1.0000x reference)
import functools

import jax
import jax.numpy as jnp
import numpy as np
from jax import lax
from jax.experimental import pallas as pl
from jax.experimental.pallas import tpu as pltpu
from jax.experimental.pallas import tpu_sc as plsc

VOCAB = 1000000
D_MODEL = 64
SEQ = 200
NSEQ = 4096

NC = 2
NS = 16
NW = NC * NS

ITERS = NSEQ // NW
SPLITS = ((0, 104), (104, 96))
NBUF = 4


def _positional_encoding_np(length, d_model):
    depth = d_model / 2
    depths = np.arange(depth)[np.newaxis, :] / depth
    angle_rads = np.arange(length)[:, np.newaxis] / 10000 ** depths
    return np.concatenate(
        [np.sin(angle_rads), np.cos(angle_rads)], axis=-1
    ).astype(np.float32)


_mesh = plsc.VectorSubcoreMesh(core_axis_name="c", subcore_axis_name="s")


@functools.partial(
    pl.kernel,
    mesh=_mesh,
    out_type=jax.ShapeDtypeStruct((NSEQ, SEQ, D_MODEL), jnp.float32),
    scratch_types=[
        pltpu.VMEM((ITERS, SEQ), jnp.int32),
        pltpu.VMEM((NBUF, SEQ, D_MODEL), jnp.float32),
        [pltpu.SemaphoreType.DMA] * NBUF,
        [pltpu.SemaphoreType.DMA] * NBUF,
    ],
    compiler_params=pltpu.CompilerParams(use_tc_tiling_on_sc=False),
)
def _gather_kernel(table_hbm, idx_hbm, out_hbm, idx_all, rows_v, sem_g, sem_o):
    wid = lax.axis_index("s") * NC + lax.axis_index("c")
    seq0 = wid * ITERS

    pltpu.sync_copy(idx_hbm.at[pl.ds(seq0, ITERS)], idx_all)

    def issue_gathers(c, b):
        for off, ln in SPLITS:
            pltpu.async_copy(
                table_hbm.at[idx_all.at[c, pl.ds(off, ln)]],
                rows_v.at[b, pl.ds(off, ln)],
                sem_g[b],
            )

    def drain_gathers(b):
        for off, ln in SPLITS:
            pltpu.make_async_copy(
                table_hbm.at[idx_all.at[0, pl.ds(off, ln)]],
                rows_v.at[b, pl.ds(off, ln)],
                sem_g[b],
            ).wait()

    def issue_out(c, b):
        pltpu.async_copy(rows_v.at[b], out_hbm.at[seq0 + c], sem_o[b])

    def drain_out(b):
        pltpu.make_async_copy(
            rows_v.at[b], out_hbm.at[seq0], sem_o[b]
        ).wait()

    for c in range(NBUF - 1):
        issue_gathers(c, c)

    def step(s, _):
        for j in range(NBUF):
            c = NBUF * s + j
            b = j
            nb = (j + NBUF - 1) % NBUF

            drain_gathers(b)
            issue_out(c, b)

            if j == 0:

                @pl.when(s > 0)
                def _():
                    drain_out(nb)
                    issue_gathers(c + NBUF - 1, nb)

                @pl.when(s == 0)
                def _():
                    issue_gathers(c + NBUF - 1, nb)
            else:

                @pl.when(s < ITERS // NBUF - 1)
                def _():
                    drain_out(nb)
                    issue_gathers(c + NBUF - 1, nb)

        return 0

    lax.fori_loop(0, ITERS // NBUF, step, 0)

    for b in range(NBUF):
        drain_out(b)


_PE = _positional_encoding_np(SEQ, D_MODEL)


def kernel(x, table):
    table8 = table * 8.0
    emb = _gather_kernel(table8, x.astype(jnp.int32))
    return emb + jnp.asarray(_PE)[None, :, :]


# speedup vs baseline: 1.4558x; 1.4558x over previous
import functools

import jax
import jax.numpy as jnp
import numpy as np
from jax import lax
from jax.experimental import pallas as pl
from jax.experimental.pallas import tpu as pltpu
from jax.experimental.pallas import tpu_sc as plsc

VOCAB = 1000000
D_MODEL = 64
SEQ = 200
NSEQ = 4096

NC = 2
NS = 16
NW = NC * NS

ITERS = NSEQ // NW
NBUF = 4


def _positional_encoding_np(length, d_model):
    depth = d_model / 2
    depths = np.arange(depth)[np.newaxis, :] / depth
    angle_rads = np.arange(length)[:, np.newaxis] / 10000 ** depths
    return np.concatenate(
        [np.sin(angle_rads), np.cos(angle_rads)], axis=-1
    ).astype(np.float32)


_mesh = plsc.VectorSubcoreMesh(core_axis_name="c", subcore_axis_name="s")


@functools.partial(
    pl.kernel,
    mesh=_mesh,
    out_type=jax.ShapeDtypeStruct((NSEQ, SEQ, D_MODEL), jnp.float32),
    scratch_types=[
        pltpu.VMEM((ITERS, SEQ), jnp.int32),
        pltpu.VMEM((NBUF, SEQ, D_MODEL), jnp.float32),
        pltpu.VMEM((SEQ, D_MODEL), jnp.float32),
        [pltpu.SemaphoreType.DMA] * NBUF,
        [pltpu.SemaphoreType.DMA] * NBUF,
    ],
    compiler_params=pltpu.CompilerParams(use_tc_tiling_on_sc=False),
)
def _emb_kernel(table_hbm, idx_hbm, pe_hbm, out_hbm, idx_all, rows_v, pe_v,
                sem_g, sem_o):
    wid = lax.axis_index("s") * NC + lax.axis_index("c")
    seq0 = wid * ITERS

    pltpu.sync_copy(pe_hbm, pe_v)
    pltpu.sync_copy(idx_hbm.at[pl.ds(seq0, ITERS)], idx_all)

    def issue_gathers(c, b):
        pltpu.async_copy(
            table_hbm.at[idx_all.at[c]], rows_v.at[b], sem_g[b]
        )

    def drain_gathers(b):
        pltpu.make_async_copy(
            table_hbm.at[idx_all.at[0]], rows_v.at[b], sem_g[b]
        ).wait()

    def issue_out(c, b):
        pltpu.async_copy(rows_v.at[b], out_hbm.at[seq0 + c], sem_o[b])

    def drain_out(b):
        pltpu.make_async_copy(
            rows_v.at[b], out_hbm.at[seq0], sem_o[b]
        ).wait()

    def compute(b):
        def body(i, _):
            for d in range(D_MODEL // 16):
                sl = pl.ds(d * 16, 16)
                rows_v[b, i, sl] = rows_v[b, i, sl] * 8.0 + pe_v[i, sl]
            return 0

        lax.fori_loop(0, SEQ, body, 0)

    for c in range(NBUF - 1):
        issue_gathers(c, c)

    def step(s, _):
        for j in range(NBUF):
            c = NBUF * s + j
            b = j
            nb = (j + NBUF - 1) % NBUF

            drain_gathers(b)
            compute(b)
            issue_out(c, b)

            if j == 0:

                @pl.when(s > 0)
                def _():
                    drain_out(nb)
                    issue_gathers(c + NBUF - 1, nb)

                @pl.when(s == 0)
                def _():
                    issue_gathers(c + NBUF - 1, nb)
            else:

                @pl.when(s < ITERS // NBUF - 1)
                def _():
                    drain_out(nb)
                    issue_gathers(c + NBUF - 1, nb)

        return 0

    lax.fori_loop(0, ITERS // NBUF, step, 0)

    for b in range(NBUF):
        drain_out(b)


_PE = _positional_encoding_np(SEQ, D_MODEL)


def kernel(x, table):
    pe = jnp.asarray(_PE)
    return _emb_kernel(table, x.astype(jnp.int32), pe)
